# Initial kernel scaffold; baseline (speedup 1.0000x reference)
import jax
import jax.numpy as jnp
from jax.experimental import pallas as pl


def kernel(last_assignment, emb_table, W, b):
    raise NotImplementedError("write your pallas kernel here")


# speedup vs baseline: 118.1518x; 118.1518x over previous
import jax
import jax.numpy as jnp
import numpy as np
from jax import lax
from jax.experimental import pallas as pl

NUM_QUBITS = 4096
NUM_CORES = 16
CORE_SIZE = 512
HIDDEN = 128
CHUNK = 512
NCHUNKS = NUM_QUBITS // CHUNK
MINF = -3.0e38


def _body(a_ref, embT_ref, Wt_ref, b_ref, tri_ref, out_ref):
    tri = tri_ref[...]
    carry = jnp.zeros((NUM_CORES, 1), jnp.float32)
    accs = [jnp.full((HIDDEN, 1), MINF, jnp.float32) for _ in range(NUM_CORES)]
    for k in range(NCHUNKS):
        a_chunk = a_ref[0, :, k * CHUNK:(k + 1) * CHUNK]
        iota_c = lax.broadcasted_iota(jnp.int32, (NUM_CORES, CHUNK), 0)
        oh = (iota_c == a_chunk).astype(jnp.float32)
        rank = jnp.dot(oh, tri, preferred_element_type=jnp.float32) + carry
        rank_q = jnp.sum(rank * oh, axis=0, keepdims=True)
        inc = rank_q < float(CORE_SIZE)
        carry = carry + jnp.sum(oh, axis=1, keepdims=True)
        chunk = embT_ref[:, k * CHUNK:(k + 1) * CHUNK]
        for c in range(NUM_CORES):
            m = jnp.logical_and(a_chunk == c, inc)
            masked = jnp.where(m, chunk, MINF)
            accs[c] = jnp.maximum(accs[c], jnp.max(masked, axis=1, keepdims=True))
    accl = jnp.concatenate(accs, axis=1)
    eye = (lax.broadcasted_iota(jnp.int32, (NUM_CORES, NUM_CORES), 0)
           == lax.broadcasted_iota(jnp.int32, (NUM_CORES, NUM_CORES), 1))
    carry_rep = jnp.broadcast_to(carry, (NUM_CORES, NUM_CORES))
    counts_lane = jnp.sum(jnp.where(eye, carry_rep, 0.0), axis=0, keepdims=True)
    g = jnp.where(counts_lane < float(CORE_SIZE), 0.0, MINF)
    adjusted = jnp.maximum(accl, g)
    s_col = jnp.sum(adjusted, axis=1, keepdims=True)
    y = jnp.dot(Wt_ref[...], s_col, preferred_element_type=jnp.float32)
    y = y * (1.0 / NUM_CORES) + b_ref[...]
    out_ref[0] = jnp.broadcast_to(y, (HIDDEN, NUM_CORES))


def kernel(last_assignment, emb_table, W, b):
    Bs = last_assignment.shape[0]
    a3 = last_assignment.reshape(Bs, 1, NUM_QUBITS)
    embT = emb_table[:NUM_QUBITS].T
    Wt = W.T
    b_col = b[:, None]
    tri = jnp.asarray(np.triu(np.ones((CHUNK, CHUNK), np.float32), 1))
    out = pl.pallas_call(
        _body,
        grid=(Bs,),
        in_specs=[
            pl.BlockSpec((1, 1, NUM_QUBITS), lambda i: (i, 0, 0)),
            pl.BlockSpec((HIDDEN, NUM_QUBITS), lambda i: (0, 0)),
            pl.BlockSpec((HIDDEN, HIDDEN), lambda i: (0, 0)),
            pl.BlockSpec((HIDDEN, 1), lambda i: (0, 0)),
            pl.BlockSpec((CHUNK, CHUNK), lambda i: (0, 0)),
        ],
        out_specs=pl.BlockSpec((1, HIDDEN, NUM_CORES), lambda i: (i, 0, 0)),
        out_shape=jax.ShapeDtypeStruct((Bs, HIDDEN, NUM_CORES), jnp.float32),
    )(a3, embT, Wt, b_col, tri)
    return jnp.swapaxes(out, 1, 2)
